# baseline (speedup 1.0000x reference)
import functools

import jax
import jax.numpy as jnp
from jax import lax
from jax.experimental import pallas as pl
from jax.experimental.pallas import tpu as pltpu
from jax.experimental.pallas import tpu_sc as plsc

DIM = 64
CAP = 500000
NQ = 32
KK = 8
BLK = 4096
GRID = (CAP + BLK - 1) // BLK

_NEG_INF = float("-inf")
_BIG_I = 2**30


def _topk_body(q_ref, kt_ref, scores_out, idx_out, rv_ref, ri_ref):
    t = pl.program_id(0)

    @pl.when(t == 0)
    def _init():
        rv_ref[...] = jnp.full((NQ, KK), _NEG_INF, jnp.float32)
        ri_ref[...] = jnp.full((NQ, KK), _BIG_I, jnp.int32)

    q = q_ref[...]
    qn = q / jnp.maximum(
        jnp.sqrt(jnp.sum(q * q, axis=1, keepdims=True)), 1e-12)
    kt = kt_ref[...]
    ss = jnp.sum(kt * kt, axis=0, keepdims=True)
    kn = kt / jnp.maximum(jnp.sqrt(ss), 1e-12)
    simn = lax.dot_general(
        qn, kn, (((1,), (0,)), ((), ())),
        preferred_element_type=jnp.float32)

    col = lax.broadcasted_iota(jnp.int32, (NQ, BLK), 1) + t * BLK
    simn = jnp.where(col < CAP, simn, _NEG_INF)

    comb_v = jnp.concatenate([rv_ref[...], simn], axis=1)
    comb_i = jnp.concatenate([ri_ref[...], col], axis=1)

    vals, idxs = [], []
    for _ in range(KK):
        m = jnp.max(comb_v, axis=1)
        eq = comb_v == m[:, None]
        ci = jnp.min(jnp.where(eq, comb_i, _BIG_I), axis=1)
        vals.append(m)
        idxs.append(ci)
        comb_v = jnp.where(comb_i == ci[:, None], _NEG_INF, comb_v)
    rv = jnp.stack(vals, axis=1)
    ri = jnp.stack(idxs, axis=1)
    rv_ref[...] = rv
    ri_ref[...] = ri

    @pl.when(t == GRID - 1)
    def _fin():
        scores_out[...] = rv
        idx_out[...] = ri


_topk_call = pl.pallas_call(
    _topk_body,
    grid=(GRID,),
    in_specs=[
        pl.BlockSpec((NQ, DIM), lambda t: (0, 0)),
        pl.BlockSpec((DIM, BLK), lambda t: (0, t)),
    ],
    out_specs=[
        pl.BlockSpec((NQ, KK), lambda t: (0, 0)),
        pl.BlockSpec((NQ, KK), lambda t: (0, 0)),
    ],
    out_shape=[
        jax.ShapeDtypeStruct((NQ, KK), jnp.float32),
        jax.ShapeDtypeStruct((NQ, KK), jnp.int32),
    ],
    scratch_shapes=[
        pltpu.VMEM((NQ, KK), jnp.float32),
        pltpu.VMEM((NQ, KK), jnp.int32),
    ],
    compiler_params=pltpu.CompilerParams(
        dimension_semantics=("arbitrary",)),
)


_NC, _NS = 2, 16
_NW = _NC * _NS
ROWS = NQ * KK
RPW = ROWS // _NW


@functools.cache
def _make_sc_gather():
    mesh = plsc.VectorSubcoreMesh(core_axis_name="c", subcore_axis_name="s")

    @functools.partial(
        pl.kernel,
        mesh=mesh,
        out_type=[
            jax.ShapeDtypeStruct((_NW, DIM, RPW), jnp.float32),
            jax.ShapeDtypeStruct((_NW, DIM, RPW), jnp.float32),
        ],
        scratch_types=[
            pltpu.VMEM((RPW,), jnp.int32),
            pltpu.VMEM((DIM, 16), jnp.float32),
            pltpu.VMEM((DIM, 16), jnp.float32),
            pltpu.SemaphoreType.DMA,
        ],
        compiler_params=pltpu.CompilerParams(use_tc_tiling_on_sc=False),
    )
    def _sc_gather(kt_hbm, vt_hbm, idx_hbm, outk_hbm, outv_hbm,
                   idx_v, ktr, vtr, sem):
        wid = lax.axis_index("s") * _NC + lax.axis_index("c")
        base = wid * RPW
        pltpu.sync_copy(idx_hbm.at[pl.ds(base, RPW)], idx_v)

        @pl.loop(0, DIM, step=8)
        def gather_chunk(d0):
            for dd in range(8):
                d = d0 + dd
                pltpu.async_copy(
                    kt_hbm.at[d].at[idx_v], ktr.at[d, pl.ds(0, RPW)], sem)
                pltpu.async_copy(
                    vt_hbm.at[d].at[idx_v], vtr.at[d, pl.ds(0, RPW)], sem)
            for dd in range(8):
                d = d0 + dd
                pltpu.make_async_copy(
                    kt_hbm.at[d].at[idx_v], ktr.at[d, pl.ds(0, RPW)],
                    sem).wait()
                pltpu.make_async_copy(
                    vt_hbm.at[d].at[idx_v], vtr.at[d, pl.ds(0, RPW)],
                    sem).wait()

        acc = ktr[0, :] * ktr[0, :]
        for d in range(1, DIM):
            x = ktr[d, :]
            acc = acc + x * x
        x = jnp.maximum(acc, 1e-30)
        s = 0.5 * (x + 1.0)
        for _ in range(15):
            s = 0.5 * (s + x / s)
        y = 1.0 / jnp.maximum(s, 1e-12)
        for d in range(DIM):
            ktr[d, :] = ktr[d, :] * y

        pltpu.sync_copy(ktr.at[:, pl.ds(0, RPW)], outk_hbm.at[wid])
        pltpu.sync_copy(vtr.at[:, pl.ds(0, RPW)], outv_hbm.at[wid])

    return _sc_gather


def kernel(k, v, query, top_k):
    del top_k
    kt = jnp.swapaxes(k, 0, 1)
    vt = jnp.swapaxes(v, 0, 1)
    scores, idx = _topk_call(query, kt)
    outk_t, outv_t = _make_sc_gather()(kt, vt, idx.reshape(-1))
    outk = jnp.swapaxes(outk_t, 1, 2)
    outv = jnp.swapaxes(outv_t, 1, 2)
    return outk, outv, scores


# speedup vs baseline: 4.4320x; 4.4320x over previous
import functools

import jax
import jax.numpy as jnp
from jax import lax
from jax.experimental import pallas as pl
from jax.experimental.pallas import tpu as pltpu
from jax.experimental.pallas import tpu_sc as plsc

DIM = 64
CAP = 500000
NQ = 32
KK = 8
BLK = 4096
GRID = (CAP + BLK - 1) // BLK

_NEG_INF = float("-inf")
_BIG_I = 2**30


def _topk_body(q_ref, kt_ref, scores_out, idx_out, rv_ref, ri_ref):
    t = pl.program_id(0)

    @pl.when(t == 0)
    def _init():
        rv_ref[...] = jnp.full((NQ, KK), _NEG_INF, jnp.float32)
        ri_ref[...] = jnp.full((NQ, KK), _BIG_I, jnp.int32)

    q = q_ref[...]
    qn = q / jnp.maximum(
        jnp.sqrt(jnp.sum(q * q, axis=1, keepdims=True)), 1e-12)
    kt = kt_ref[...]
    ss = jnp.sum(kt * kt, axis=0, keepdims=True)
    kn = kt / jnp.maximum(jnp.sqrt(ss), 1e-12)
    simn = lax.dot_general(
        qn, kn, (((1,), (0,)), ((), ())),
        preferred_element_type=jnp.float32)

    col = lax.broadcasted_iota(jnp.int32, (NQ, BLK), 1) + t * BLK
    simn = jnp.where(col < CAP, simn, _NEG_INF)

    comb_v = jnp.concatenate([rv_ref[...], simn], axis=1)
    comb_i = jnp.concatenate([ri_ref[...], col], axis=1)

    vals, idxs = [], []
    for _ in range(KK):
        m = jnp.max(comb_v, axis=1)
        eq = comb_v == m[:, None]
        ci = jnp.min(jnp.where(eq, comb_i, _BIG_I), axis=1)
        vals.append(m)
        idxs.append(ci)
        comb_v = jnp.where(comb_i == ci[:, None], _NEG_INF, comb_v)
    rv = jnp.stack(vals, axis=1)
    ri = jnp.stack(idxs, axis=1)
    rv_ref[...] = rv
    ri_ref[...] = ri

    @pl.when(t == GRID - 1)
    def _fin():
        scores_out[...] = rv
        idx_out[...] = ri


_topk_call = pl.pallas_call(
    _topk_body,
    grid=(GRID,),
    in_specs=[
        pl.BlockSpec((NQ, DIM), lambda t: (0, 0)),
        pl.BlockSpec((DIM, BLK), lambda t: (0, t)),
    ],
    out_specs=[
        pl.BlockSpec((NQ, KK), lambda t: (0, 0)),
        pl.BlockSpec((NQ, KK), lambda t: (0, 0)),
    ],
    out_shape=[
        jax.ShapeDtypeStruct((NQ, KK), jnp.float32),
        jax.ShapeDtypeStruct((NQ, KK), jnp.int32),
    ],
    scratch_shapes=[
        pltpu.VMEM((NQ, KK), jnp.float32),
        pltpu.VMEM((NQ, KK), jnp.int32),
    ],
    compiler_params=pltpu.CompilerParams(
        dimension_semantics=("arbitrary",)),
)


_NC, _NS = 2, 16
_NW = _NC * _NS
ROWS = NQ * KK
RPW = ROWS // _NW


@functools.cache
def _make_sc_gather():
    mesh = plsc.VectorSubcoreMesh(core_axis_name="c", subcore_axis_name="s")

    @functools.partial(
        pl.kernel,
        mesh=mesh,
        out_type=[
            jax.ShapeDtypeStruct((ROWS, DIM), jnp.float32),
            jax.ShapeDtypeStruct((ROWS, DIM), jnp.float32),
        ],
        scratch_types=[
            pltpu.VMEM((RPW,), jnp.int32),
            pltpu.VMEM((RPW, DIM), jnp.float32),
            pltpu.VMEM((RPW, DIM), jnp.float32),
            pltpu.SemaphoreType.DMA,
        ],
        compiler_params=pltpu.CompilerParams(use_tc_tiling_on_sc=False),
    )
    def _sc_gather(k_hbm, v_hbm, idx_hbm, outk_hbm, outv_hbm,
                   idx_v, krows, vrows, sem):
        wid = lax.axis_index("s") * _NC + lax.axis_index("c")
        base = wid * RPW
        pltpu.sync_copy(idx_hbm.at[pl.ds(base, RPW)], idx_v)
        pltpu.async_copy(k_hbm.at[idx_v], krows, sem).wait()
        pltpu.async_copy(v_hbm.at[idx_v], vrows, sem).wait()

        for r in range(RPW):
            x0 = krows[r, pl.ds(0, 16)]
            acc = x0 * x0
            for c in range(1, DIM // 16):
                x = krows[r, pl.ds(c * 16, 16)]
                acc = acc + x * x
            lanes = lax.iota(jnp.int32, 16)
            dnums = lax.GatherDimensionNumbers(
                offset_dims=(), collapsed_slice_dims=(0,),
                start_index_map=(0,))
            sv = acc
            for h in (1, 2, 4, 8):
                sv = sv + lax.gather(
                    sv, (lanes ^ h)[:, None], dnums, slice_sizes=(1,),
                    mode=lax.GatherScatterMode.PROMISE_IN_BOUNDS)
            x = jnp.maximum(sv, 1e-30)
            s = 0.5 * (x + 1.0)
            for _ in range(15):
                s = 0.5 * (s + x / s)
            y = 1.0 / jnp.maximum(s, 1e-12)
            for c in range(DIM // 16):
                sl = pl.ds(c * 16, 16)
                krows[r, sl] = krows[r, sl] * y

        pltpu.sync_copy(krows, outk_hbm.at[pl.ds(base, RPW)])
        pltpu.sync_copy(vrows, outv_hbm.at[pl.ds(base, RPW)])

    return _sc_gather


def kernel(k, v, query, top_k):
    del top_k
    kt = jnp.swapaxes(k, 0, 1)
    scores, idx = _topk_call(query, kt)
    outk, outv = _make_sc_gather()(k, v, idx.reshape(-1))
    return (outk.reshape(NQ, KK, DIM),
            outv.reshape(NQ, KK, DIM),
            scores)


# speedup vs baseline: 5.4241x; 1.2238x over previous
import functools

import jax
import jax.numpy as jnp
from jax import lax
from jax.experimental import pallas as pl
from jax.experimental.pallas import tpu as pltpu
from jax.experimental.pallas import tpu_sc as plsc

DIM = 64
CAP = 500000
NQ = 32
KK = 8
BLK = 8192
GRID = (CAP + BLK - 1) // BLK

_NEG_INF = float("-inf")
_BIG_I = 2**30


_BIG_F = float(2**25)


def _topk_body(q_ref, kt_ref, scores_out, idx_out, rv_ref, ri_ref, qn_ref):
    t = pl.program_id(0)

    @pl.when(t == 0)
    def _init():
        rv_ref[...] = jnp.full((NQ, KK), _NEG_INF, jnp.float32)
        ri_ref[...] = jnp.full((NQ, KK), _BIG_F, jnp.float32)
        q = q_ref[...]
        qn_ref[...] = q / jnp.maximum(
            jnp.sqrt(jnp.sum(q * q, axis=1, keepdims=True)), 1e-12)

    qn = qn_ref[...]
    kt = kt_ref[...]
    ss = jnp.sum(kt * kt, axis=0, keepdims=True)
    kn = kt / jnp.maximum(jnp.sqrt(ss), 1e-12)
    simn = lax.dot_general(
        qn, kn, (((1,), (0,)), ((), ())),
        preferred_element_type=jnp.float32)

    col = (lax.broadcasted_iota(jnp.int32, (NQ, BLK), 1).astype(jnp.float32)
           + jnp.float32(t * BLK))
    simn = jnp.where(col < float(CAP), simn, _NEG_INF)

    comb_v = jnp.concatenate([rv_ref[...], simn], axis=1)
    comb_i = jnp.concatenate([ri_ref[...], col], axis=1)

    vals, idxs = [], []
    for _ in range(KK):
        m = jnp.max(comb_v, axis=1)
        eq = comb_v == m[:, None]
        ci = jnp.min(jnp.where(eq, comb_i, _BIG_F), axis=1)
        vals.append(m)
        idxs.append(ci)
        comb_v = jnp.where(comb_i == ci[:, None], _NEG_INF, comb_v)
    rv = jnp.stack(vals, axis=1)
    ri = jnp.stack(idxs, axis=1)
    rv_ref[...] = rv
    ri_ref[...] = ri

    @pl.when(t == GRID - 1)
    def _fin():
        scores_out[...] = rv
        idx_out[...] = ri.astype(jnp.int32)


_topk_call = pl.pallas_call(
    _topk_body,
    grid=(GRID,),
    in_specs=[
        pl.BlockSpec((NQ, DIM), lambda t: (0, 0)),
        pl.BlockSpec((DIM, BLK), lambda t: (0, t)),
    ],
    out_specs=[
        pl.BlockSpec((NQ, KK), lambda t: (0, 0)),
        pl.BlockSpec((NQ, KK), lambda t: (0, 0)),
    ],
    out_shape=[
        jax.ShapeDtypeStruct((NQ, KK), jnp.float32),
        jax.ShapeDtypeStruct((NQ, KK), jnp.int32),
    ],
    scratch_shapes=[
        pltpu.VMEM((NQ, KK), jnp.float32),
        pltpu.VMEM((NQ, KK), jnp.float32),
        pltpu.VMEM((NQ, DIM), jnp.float32),
    ],
    compiler_params=pltpu.CompilerParams(
        dimension_semantics=("arbitrary",)),
)


_NC, _NS = 2, 16
_NW = _NC * _NS
ROWS = NQ * KK
RPW = ROWS // _NW


@functools.cache
def _make_sc_gather():
    mesh = plsc.VectorSubcoreMesh(core_axis_name="c", subcore_axis_name="s")

    @functools.partial(
        pl.kernel,
        mesh=mesh,
        out_type=[
            jax.ShapeDtypeStruct((ROWS, DIM), jnp.float32),
            jax.ShapeDtypeStruct((ROWS, DIM), jnp.float32),
        ],
        scratch_types=[
            pltpu.VMEM((RPW,), jnp.int32),
            pltpu.VMEM((RPW, DIM), jnp.float32),
            pltpu.VMEM((RPW, DIM), jnp.float32),
            pltpu.SemaphoreType.DMA,
        ],
        compiler_params=pltpu.CompilerParams(use_tc_tiling_on_sc=False),
    )
    def _sc_gather(k_hbm, v_hbm, idx_hbm, outk_hbm, outv_hbm,
                   idx_v, krows, vrows, sem):
        wid = lax.axis_index("s") * _NC + lax.axis_index("c")
        base = wid * RPW
        pltpu.sync_copy(idx_hbm.at[pl.ds(base, RPW)], idx_v)
        pltpu.async_copy(k_hbm.at[idx_v], krows, sem).wait()
        pltpu.async_copy(v_hbm.at[idx_v], vrows, sem).wait()

        for r in range(RPW):
            x0 = krows[r, pl.ds(0, 16)]
            acc = x0 * x0
            for c in range(1, DIM // 16):
                x = krows[r, pl.ds(c * 16, 16)]
                acc = acc + x * x
            lanes = lax.iota(jnp.int32, 16)
            dnums = lax.GatherDimensionNumbers(
                offset_dims=(), collapsed_slice_dims=(0,),
                start_index_map=(0,))
            sv = acc
            for h in (1, 2, 4, 8):
                sv = sv + lax.gather(
                    sv, (lanes ^ h)[:, None], dnums, slice_sizes=(1,),
                    mode=lax.GatherScatterMode.PROMISE_IN_BOUNDS)
            x = jnp.maximum(sv, 1e-30)
            s = 0.5 * (x + 1.0)
            for _ in range(15):
                s = 0.5 * (s + x / s)
            y = 1.0 / jnp.maximum(s, 1e-12)
            for c in range(DIM // 16):
                sl = pl.ds(c * 16, 16)
                krows[r, sl] = krows[r, sl] * y

        pltpu.sync_copy(krows, outk_hbm.at[pl.ds(base, RPW)])
        pltpu.sync_copy(vrows, outv_hbm.at[pl.ds(base, RPW)])

    return _sc_gather


def kernel(k, v, query, top_k):
    del top_k
    kt = jnp.swapaxes(k, 0, 1)
    scores, idx = _topk_call(query, kt)
    outk, outv = _make_sc_gather()(k, v, idx.reshape(-1))
    return (outk.reshape(NQ, KK, DIM),
            outv.reshape(NQ, KK, DIM),
            scores)
